# baseline (speedup 1.0000x reference)
import jax
import jax.numpy as jnp
from jax import lax
from jax.experimental import pallas as pl
from jax.experimental.pallas import tpu as pltpu
from jax.experimental.pallas import tpu_sc as plsc

_K = 8
_E = 64
_BM = 512
_NSC = 32
_NEG = -3.0e38


def _gate_body(x_ref, w1a_ref, w1b_ref, w1c_ref, w1d_ref, b1_ref, w2_ref,
               b2_ref, pt_ref):
    ht = None
    for j, w1_ref in enumerate((w1a_ref, w1b_ref, w1c_ref, w1d_ref)):
        part = jax.lax.dot_general(
            w1_ref[...], x_ref[:, j, :],
            (((1,), (1,)), ((), ())),
            preferred_element_type=jnp.float32)
        ht = part if ht is None else ht + part
    ht = jnp.maximum(ht + b1_ref[...], 0.0)
    logits = jax.lax.dot_general(
        w2_ref[...], ht, (((1,), (0,)), ((), ())),
        preferred_element_type=jnp.float32)
    logits = logits + b2_ref[...]
    m = jnp.max(logits, axis=0, keepdims=True)
    e = jnp.exp(logits - m)
    pt_ref[...] = e / jnp.sum(e, axis=0, keepdims=True)


def _topk_body(pt_hbm, r_hbm, idx_hbm, pv, rv, iv, sem):
    rw = pv.shape[1]
    wid = lax.axis_index("s") * 2 + lax.axis_index("c")
    base = wid * rw
    cp = pltpu.async_copy(pt_hbm.at[:, pl.ds(base, rw)], pv, sem)

    def _zero(r, _):
        for c in range(_E // 16):
            rv[r, pl.ds(c * 16, 16)] = jnp.zeros((16,), jnp.float32)
        return _
    lax.fori_loop(0, rw, _zero, 0)
    cp.wait()

    lane = lax.iota(jnp.int32, 16)
    _NG = 4
    for g0 in range(0, rw // 16, _NG):

        def _insert(e, carry):
            out = []
            for q in range(_NG):
                t = list(carry[q * 2 * _K:q * 2 * _K + _K])
                ti = list(carry[q * 2 * _K + _K:(q + 1) * 2 * _K])
                v = pv[e, pl.ds((g0 + q) * 16, 16)]
                vi = jnp.full((16,), 0, jnp.int32) + e
                for j in range(_K):
                    c = v > t[j]
                    t[j], v = jnp.where(c, v, t[j]), jnp.where(c, t[j], v)
                    ti[j], vi = (jnp.where(c, vi, ti[j]),
                                 jnp.where(c, ti[j], vi))
                out.extend(t + ti)
            return tuple(out)

        init = tuple(
            jnp.full((16,), _NEG, jnp.float32) if (i // _K) % 2 == 0
            else jnp.zeros((16,), jnp.int32)
            for i in range(2 * _K * _NG))
        res = lax.fori_loop(0, _E, _insert, init)
        for q in range(_NG):
            rows = (g0 + q) * 16 + lane
            t = res[q * 2 * _K:q * 2 * _K + _K]
            ti = res[q * 2 * _K + _K:(q + 1) * 2 * _K]
            for j in range(_K):
                plsc.store_scatter(iv, [rows, jnp.full((16,), j, jnp.int32)],
                                   ti[j])
                plsc.store_scatter(rv, [rows, ti[j]], t[j])

    pltpu.sync_copy(rv, r_hbm.at[pl.ds(base, rw), :])
    pltpu.sync_copy(iv, idx_hbm.at[pl.ds(base, rw), :])


def kernel(tokens, W1, b1, W2, b2):
    B, G, Dg = tokens.shape
    H = W1.shape[0]
    probsT = pl.pallas_call(
        _gate_body,
        grid=(B // _BM,),
        in_specs=[
            pl.BlockSpec((_BM, G, Dg), lambda i: (i, 0, 0)),
            pl.BlockSpec((H, Dg), lambda i: (0, 0)),
            pl.BlockSpec((H, Dg), lambda i: (0, 1)),
            pl.BlockSpec((H, Dg), lambda i: (0, 2)),
            pl.BlockSpec((H, Dg), lambda i: (0, 3)),
            pl.BlockSpec((H, 1), lambda i: (0, 0)),
            pl.BlockSpec((_E, H), lambda i: (0, 0)),
            pl.BlockSpec((_E, 1), lambda i: (0, 0)),
        ],
        out_specs=pl.BlockSpec((_E, _BM), lambda i: (0, i)),
        out_shape=jax.ShapeDtypeStruct((_E, B), jnp.float32),
    )(tokens, W1, W1, W1, W1, b1.reshape(H, 1), W2, b2.reshape(_E, 1))

    mesh = plsc.VectorSubcoreMesh(core_axis_name="c", subcore_axis_name="s")
    r, idx = pl.kernel(
        _topk_body,
        mesh=mesh,
        compiler_params=pltpu.CompilerParams(needs_layout_passes=False),
        out_type=[
            jax.ShapeDtypeStruct((B, _E), jnp.float32),
            jax.ShapeDtypeStruct((B, _K), jnp.int32),
        ],
        scratch_types=[
            pltpu.VMEM((_E, B // _NSC), jnp.float32),
            pltpu.VMEM((B // _NSC, _E), jnp.float32),
            pltpu.VMEM((B // _NSC, _K), jnp.int32),
            pltpu.SemaphoreType.DMA,
        ],
    )(probsT)
    return (r, idx)


# speedup vs baseline: 1.1197x; 1.1197x over previous
import jax
import jax.numpy as jnp
from jax import lax
from jax.experimental import pallas as pl
from jax.experimental.pallas import tpu as pltpu
from jax.experimental.pallas import tpu_sc as plsc

_K = 8
_E = 64
_BM = 512
_NSC = 32
_NEG = -3.0e38


def _gate_body(x_ref, w1a_ref, w1b_ref, w1c_ref, w1d_ref, b1_ref, w2_ref,
               b2_ref, pt_ref):
    ht = None
    for j, w1_ref in enumerate((w1a_ref, w1b_ref, w1c_ref, w1d_ref)):
        part = jax.lax.dot_general(
            w1_ref[...], x_ref[:, j, :],
            (((1,), (1,)), ((), ())),
            preferred_element_type=jnp.float32)
        ht = part if ht is None else ht + part
    ht = jnp.maximum(ht + b1_ref[...], 0.0)
    logits = jax.lax.dot_general(
        w2_ref[...], ht, (((1,), (0,)), ((), ())),
        preferred_element_type=jnp.float32)
    logits = logits + b2_ref[...]
    m = jnp.max(logits, axis=0, keepdims=True)
    e = jnp.exp(logits - m)
    pt_ref[...] = e / jnp.sum(e, axis=0, keepdims=True)


def _topk_body(pt_hbm, r_hbm, idx_hbm, pv, rv, iv, sem):
    rw = pv.shape[1]
    wid = lax.axis_index("s") * 2 + lax.axis_index("c")
    base = wid * rw
    cp = pltpu.async_copy(pt_hbm.at[:, pl.ds(base, rw)], pv, sem)

    def _zero(r, _):
        for c in range(_E // 16):
            rv[r, pl.ds(c * 16, 16)] = jnp.zeros((16,), jnp.float32)
        return _
    lax.fori_loop(0, rw, _zero, 0)
    cp.wait()

    lane = lax.iota(jnp.int32, 16)
    _NG = 2
    for g0 in range(0, rw // 16, _NG):

        def _insert(e, carry):
            out = []
            for q in range(_NG):
                t = list(carry[q * 2 * _K:q * 2 * _K + _K])
                ti = list(carry[q * 2 * _K + _K:(q + 1) * 2 * _K])
                v = pv[e, pl.ds((g0 + q) * 16, 16)]
                vi = jnp.full((16,), 0, jnp.int32) + e
                for j in range(_K):
                    c = v > t[j]
                    t[j], v = jnp.where(c, v, t[j]), jnp.where(c, t[j], v)
                    ti[j], vi = (jnp.where(c, vi, ti[j]),
                                 jnp.where(c, ti[j], vi))
                out.extend(t + ti)
            return tuple(out)

        init = tuple(
            jnp.full((16,), _NEG, jnp.float32) if (i // _K) % 2 == 0
            else jnp.zeros((16,), jnp.int32)
            for i in range(2 * _K * _NG))
        res = lax.fori_loop(0, _E, _insert, init)
        for q in range(_NG):
            rows = (g0 + q) * 16 + lane
            t = res[q * 2 * _K:q * 2 * _K + _K]
            ti = res[q * 2 * _K + _K:(q + 1) * 2 * _K]
            for j in range(_K):
                plsc.store_scatter(iv, [rows, jnp.full((16,), j, jnp.int32)],
                                   ti[j])
                plsc.store_scatter(rv, [rows, ti[j]], t[j])

    pltpu.sync_copy(rv, r_hbm.at[pl.ds(base, rw), :])
    pltpu.sync_copy(iv, idx_hbm.at[pl.ds(base, rw), :])


def kernel(tokens, W1, b1, W2, b2):
    B, G, Dg = tokens.shape
    H = W1.shape[0]
    probsT = pl.pallas_call(
        _gate_body,
        grid=(B // _BM,),
        in_specs=[
            pl.BlockSpec((_BM, G, Dg), lambda i: (i, 0, 0)),
            pl.BlockSpec((H, Dg), lambda i: (0, 0)),
            pl.BlockSpec((H, Dg), lambda i: (0, 1)),
            pl.BlockSpec((H, Dg), lambda i: (0, 2)),
            pl.BlockSpec((H, Dg), lambda i: (0, 3)),
            pl.BlockSpec((H, 1), lambda i: (0, 0)),
            pl.BlockSpec((_E, H), lambda i: (0, 0)),
            pl.BlockSpec((_E, 1), lambda i: (0, 0)),
        ],
        out_specs=pl.BlockSpec((_E, _BM), lambda i: (0, i)),
        out_shape=jax.ShapeDtypeStruct((_E, B), jnp.float32),
    )(tokens, W1, W1, W1, W1, b1.reshape(H, 1), W2, b2.reshape(_E, 1))

    mesh = plsc.VectorSubcoreMesh(core_axis_name="c", subcore_axis_name="s")
    r, idx = pl.kernel(
        _topk_body,
        mesh=mesh,
        compiler_params=pltpu.CompilerParams(needs_layout_passes=False),
        out_type=[
            jax.ShapeDtypeStruct((B, _E), jnp.float32),
            jax.ShapeDtypeStruct((B, _K), jnp.int32),
        ],
        scratch_types=[
            pltpu.VMEM((_E, B // _NSC), jnp.float32),
            pltpu.VMEM((B // _NSC, _E), jnp.float32),
            pltpu.VMEM((B // _NSC, _K), jnp.int32),
            pltpu.SemaphoreType.DMA,
        ],
    )(probsT)
    return (r, idx)
